# baseline (speedup 1.0000x reference)
import functools

import jax
import jax.numpy as jnp
from jax import lax
from jax.experimental import pallas as pl
from jax.experimental.pallas import tpu as pltpu
from jax.experimental.pallas import tpu_sc as plsc

POOL = 1000
KDIM = 3840
PDIM = 13824
BATCH = 1024
NTOK = 4
TOPK = 2

BCHUNK = 128
NBCH = BATCH // BCHUNK

NC = 2
NS = 16
NW = NC * NS
LANES = 16
BPW = BATCH // NW
BCH = 4
NBC = BPW // BCH
NJOB = NBC * TOPK
NSTRIP = POOL // LANES
TAIL = POOL - LANES * (POOL // LANES)
TAILOFF = POOL - LANES
NEG = float("-inf")
BIG = 2**30


def _pk_norm_body(pk_ref, out_ref):
    pk = pk_ref[...]
    sq = jnp.sum(pk * pk, axis=1, keepdims=True)
    out_ref[...] = pk * lax.rsqrt(jnp.maximum(sq, 1e-12))


def _pk_norm(prompt_key):
    return pl.pallas_call(
        _pk_norm_body,
        grid=(5,),
        in_specs=[pl.BlockSpec((200, KDIM), lambda i: (i, 0))],
        out_specs=pl.BlockSpec((200, KDIM), lambda i: (i, 0)),
        out_shape=jax.ShapeDtypeStruct((POOL, KDIM), jnp.float32),
    )(prompt_key)


def _sim_body(x_ref, pkn_ref, sim_ref):
    x = x_ref[...]
    xm = jnp.mean(x, axis=1)
    sq = jnp.sum(xm * xm, axis=1, keepdims=True)
    xn = xm * lax.rsqrt(jnp.maximum(sq, 1e-12))
    pkn = pkn_ref[...]
    sim_ref[...] = lax.dot_general(
        xn, pkn, (((1,), (1,)), ((), ())), preferred_element_type=jnp.float32
    )


def _sim(x_embed, pk_norm):
    nb = x_embed.shape[0]
    return pl.pallas_call(
        _sim_body,
        grid=(nb // BCHUNK,),
        in_specs=[
            pl.BlockSpec((BCHUNK, NTOK, KDIM), lambda i: (i, 0, 0)),
            pl.BlockSpec((POOL, KDIM), lambda i: (0, 0)),
        ],
        out_specs=pl.BlockSpec((BCHUNK, POOL), lambda i: (i, 0)),
        out_shape=jax.ShapeDtypeStruct((nb, POOL), jnp.float32),
    )(x_embed, pk_norm)


def _rsum_body(s_ref, o_ref):
    o_ref[0, 0] = jnp.sum(s_ref[...]) * (1.0 / BATCH)


def _rsum(sums):
    return pl.pallas_call(
        _rsum_body,
        out_specs=pl.BlockSpec(memory_space=pltpu.SMEM),
        out_shape=jax.ShapeDtypeStruct((1, 1), jnp.float32),
    )(sums)


_GDN = lax.GatherDimensionNumbers(
    offset_dims=(), collapsed_slice_dims=(0,), start_index_map=(0,)
)


def _shuf(x, pi):
    return lax.gather(
        x,
        pi[:, None],
        dimension_numbers=_GDN,
        slice_sizes=(1,),
        mode=lax.GatherScatterMode.PROMISE_IN_BOUNDS,
    )


def _bcast_max_minidx(m, idx, liota):
    for sh in (8, 4, 2, 1):
        pi = liota ^ sh
        pm = _shuf(m, pi)
        pidx = _shuf(idx, pi)
        better = (pm > m) | ((pm == m) & (pidx < idx))
        m = jnp.where(better, pm, m)
        idx = jnp.where(better, pidx, idx)
    return m, idx


def _row_top2(simbuf, rr):
    liota = lax.iota(jnp.int32, LANES)

    def strip(s, carry):
        m1, m2, i1v, i2v = carry
        v = simbuf[rr, pl.ds(s * LANES, LANES)]
        ids = liota + s * LANES
        gt1 = v > m1
        gt2 = v > m2
        i2v = jnp.where(gt1, i1v, jnp.where(gt2, ids, i2v))
        m2 = jnp.where(gt1, m1, jnp.where(gt2, v, m2))
        i1v = jnp.where(gt1, ids, i1v)
        m1 = jnp.where(gt1, v, m1)
        return m1, m2, i1v, i2v

    init = (
        jnp.full((LANES,), NEG),
        jnp.full((LANES,), NEG),
        jnp.zeros((LANES,), jnp.int32),
        jnp.zeros((LANES,), jnp.int32),
    )
    m1, m2, i1v, i2v = lax.fori_loop(0, NSTRIP, strip, init)
    v = simbuf[rr, pl.ds(TAILOFF, LANES)]
    v = jnp.where(liota < (LANES - TAIL), NEG, v)
    ids = liota + TAILOFF
    gt1 = v > m1
    gt2 = v > m2
    i2v = jnp.where(gt1, i1v, jnp.where(gt2, ids, i2v))
    m2 = jnp.where(gt1, m1, jnp.where(gt2, v, m2))
    i1v = jnp.where(gt1, ids, i1v)
    m1 = jnp.where(gt1, v, m1)
    vm1, vi1 = _bcast_max_minidx(m1, i1v, liota)
    in_l = i1v == vi1
    vl = jnp.where(in_l, m2, m1)
    il = jnp.where(in_l, i2v, i1v)
    vm2, vi2 = _bcast_max_minidx(vl, il, liota)
    return vm1, vi1, vm2, vi2


@functools.cache
def _topk_gather_kernel(nbatch):
    mesh = plsc.VectorSubcoreMesh(core_axis_name="c", subcore_axis_name="s")
    bpw = nbatch // NW
    nbc = bpw // BCH
    njob = nbc * TOPK

    @functools.partial(
        pl.kernel,
        mesh=mesh,
        out_type=(
            jax.ShapeDtypeStruct((nbatch, TOPK, PDIM), jnp.float32),
            jax.ShapeDtypeStruct((NW * njob, LANES), jnp.int32),
            jax.ShapeDtypeStruct((NW, LANES), jnp.float32),
        ),
        scratch_types=[
            pltpu.VMEM((BCH, POOL), jnp.float32),
            pltpu.VMEM((njob, LANES), jnp.int32),
            pltpu.VMEM((LANES,), jnp.float32),
            pltpu.VMEM((BCH, 1, PDIM), jnp.float32),
            pltpu.VMEM((BCH, 1, PDIM), jnp.float32),
            pltpu.SemaphoreType.DMA,
            pltpu.SemaphoreType.DMA,
            pltpu.SemaphoreType.DMA,
            pltpu.SemaphoreType.DMA,
        ],
    )
    def _topk_gather(sim_hbm, prompt_hbm, out_hbm, idx_hbm, sums_hbm,
                     simbuf, jobtab, saccbuf, buf_a, buf_b,
                     gsem_a, gsem_b, wsem_a, wsem_b):
        wid = lax.axis_index("s") * NC + lax.axis_index("c")
        bb = wid * bpw
        liota = lax.iota(jnp.int32, LANES)
        state = {"total": jnp.zeros((LANES,), jnp.float32)}

        def topk_chunk(c):
            pltpu.sync_copy(sim_hbm.at[pl.ds(bb + c * BCH, BCH)], simbuf)
            v1 = jnp.zeros((LANES,), jnp.int32)
            v2 = jnp.zeros((LANES,), jnp.int32)
            for rr in range(BCH):
                vm1, vi1, vm2, vi2 = _row_top2(simbuf, rr)
                v1 = jnp.where(liota == rr, vi1, v1)
                v2 = jnp.where(liota == rr, vi2, v2)
                state["total"] = state["total"] + jnp.where(
                    liota == 0, vm1 + vm2, 0.0
                )
            jobtab[2 * c + 0, :] = v1
            jobtab[2 * c + 1, :] = v2

        topk_chunk(0)
        topk_chunk(1)
        writes = []
        for j in range(njob):
            c, p = j // 2, j % 2
            buf, gsem, wsem = (
                (buf_a, gsem_a, wsem_a) if j % 2 == 0 else (buf_b, gsem_b, wsem_b)
            )
            if j >= 2:
                writes[j - 2].wait()
            gath = pltpu.async_copy(
                prompt_hbm.at[jobtab.at[j, pl.ds(0, BCH)]], buf, gsem
            )
            if p == 0 and c + 2 < nbc:
                topk_chunk(c + 2)
            gath.wait()
            writes.append(
                pltpu.async_copy(
                    buf,
                    out_hbm.at[pl.ds(bb + c * BCH, BCH), pl.ds(p, 1)],
                    wsem,
                )
            )
        saccbuf[...] = state["total"]
        pltpu.sync_copy(saccbuf, sums_hbm.at[wid])
        pltpu.sync_copy(jobtab, idx_hbm.at[pl.ds(wid * njob, njob)])
        writes[-2].wait()
        writes[-1].wait()

    return _topk_gather


HALF = BATCH // 2


def kernel(x_embed, prompt, prompt_key):
    pk_norm = _pk_norm(prompt_key)
    sims, bps, idxs, sums = [], [], [], []
    for h in range(2):
        sim_h = _sim(x_embed[h * HALF:(h + 1) * HALF], pk_norm)
        bp_h, jobs_h, sums_h = _topk_gather_kernel(HALF)(sim_h, prompt)
        nbc = HALF // NW // BCH
        idx_h = (
            jobs_h.reshape(NW, nbc, TOPK, LANES)[:, :, :, :BCH]
            .transpose(0, 1, 3, 2)
            .reshape(HALF, TOPK)
        )
        sims.append(sim_h)
        bps.append(bp_h)
        idxs.append(idx_h)
        sums.append(sums_h)
    sim = jnp.concatenate(sims, axis=0)
    batched_prompt = jnp.concatenate(bps, axis=0)
    idx = jnp.concatenate(idxs, axis=0)
    reduce_sim = _rsum(jnp.concatenate(sums, axis=0))[0, 0]
    return batched_prompt, sim, idx, reduce_sim


# speedup vs baseline: 1.4566x; 1.4566x over previous
import functools

import jax
import jax.numpy as jnp
from jax import lax
from jax.experimental import pallas as pl
from jax.experimental.pallas import tpu as pltpu
from jax.experimental.pallas import tpu_sc as plsc

POOL = 1000
KDIM = 3840
PDIM = 13824
BATCH = 1024
NTOK = 4
TOPK = 2

BCHUNK = 128
NBCH = BATCH // BCHUNK

NC = 2
NS = 16
NW = NC * NS
LANES = 16
BPW = BATCH // NW
BCH = 4
NBC = BPW // BCH
NJOB = NBC * TOPK
NSTRIP = POOL // LANES
TAIL = POOL - LANES * (POOL // LANES)
TAILOFF = POOL - LANES
NEG = float("-inf")
BIG = 2**30


def _pk_norm_body(pk_ref, out_ref):
    pk = pk_ref[...]
    sq = jnp.sum(pk * pk, axis=1, keepdims=True)
    out_ref[...] = pk * lax.rsqrt(jnp.maximum(sq, 1e-12))


def _pk_norm(prompt_key):
    return pl.pallas_call(
        _pk_norm_body,
        grid=(5,),
        in_specs=[pl.BlockSpec((200, KDIM), lambda i: (i, 0))],
        out_specs=pl.BlockSpec((200, KDIM), lambda i: (i, 0)),
        out_shape=jax.ShapeDtypeStruct((POOL, KDIM), jnp.float32),
    )(prompt_key)


def _sim_body(x_ref, pkn_ref, sim_ref):
    x = x_ref[...]
    xm = jnp.mean(x, axis=1)
    sq = jnp.sum(xm * xm, axis=1, keepdims=True)
    xn = xm * lax.rsqrt(jnp.maximum(sq, 1e-12))
    pkn = pkn_ref[...]
    sim_ref[...] = lax.dot_general(
        xn, pkn, (((1,), (1,)), ((), ())), preferred_element_type=jnp.float32
    )


def _sim(x_embed, pk_norm):
    return pl.pallas_call(
        _sim_body,
        grid=(NBCH,),
        in_specs=[
            pl.BlockSpec((BCHUNK, NTOK, KDIM), lambda i: (i, 0, 0)),
            pl.BlockSpec((POOL, KDIM), lambda i: (0, 0)),
        ],
        out_specs=pl.BlockSpec((BCHUNK, POOL), lambda i: (i, 0)),
        out_shape=jax.ShapeDtypeStruct((BATCH, POOL), jnp.float32),
    )(x_embed, pk_norm)


def _rsum_body(s_ref, o_ref):
    o_ref[0, 0] = jnp.sum(s_ref[...]) * (1.0 / BATCH)


def _rsum(sums):
    return pl.pallas_call(
        _rsum_body,
        out_specs=pl.BlockSpec(memory_space=pltpu.SMEM),
        out_shape=jax.ShapeDtypeStruct((1, 1), jnp.float32),
    )(sums)


_GDN = lax.GatherDimensionNumbers(
    offset_dims=(), collapsed_slice_dims=(0,), start_index_map=(0,)
)


def _shuf(x, pi):
    return lax.gather(
        x,
        pi[:, None],
        dimension_numbers=_GDN,
        slice_sizes=(1,),
        mode=lax.GatherScatterMode.PROMISE_IN_BOUNDS,
    )


def _bcast_max_minidx(m, idx, liota):
    for sh in (8, 4, 2, 1):
        pi = liota ^ sh
        pm = _shuf(m, pi)
        pidx = _shuf(idx, pi)
        better = (pm > m) | ((pm == m) & (pidx < idx))
        m = jnp.where(better, pm, m)
        idx = jnp.where(better, pidx, idx)
    return m, idx


def _row_top2(simbuf, rr):
    liota = lax.iota(jnp.int32, LANES)

    def strip(s, carry):
        m1, m2, i1v, i2v = carry
        v = simbuf[rr, pl.ds(s * LANES, LANES)]
        ids = liota + s * LANES
        gt1 = v > m1
        gt2 = v > m2
        i2v = jnp.where(gt1, i1v, jnp.where(gt2, ids, i2v))
        m2 = jnp.where(gt1, m1, jnp.where(gt2, v, m2))
        i1v = jnp.where(gt1, ids, i1v)
        m1 = jnp.where(gt1, v, m1)
        return m1, m2, i1v, i2v

    init = (
        jnp.full((LANES,), NEG),
        jnp.full((LANES,), NEG),
        jnp.zeros((LANES,), jnp.int32),
        jnp.zeros((LANES,), jnp.int32),
    )
    m1, m2, i1v, i2v = lax.fori_loop(0, NSTRIP, strip, init)
    v = simbuf[rr, pl.ds(TAILOFF, LANES)]
    v = jnp.where(liota < (LANES - TAIL), NEG, v)
    ids = liota + TAILOFF
    gt1 = v > m1
    gt2 = v > m2
    i2v = jnp.where(gt1, i1v, jnp.where(gt2, ids, i2v))
    m2 = jnp.where(gt1, m1, jnp.where(gt2, v, m2))
    i1v = jnp.where(gt1, ids, i1v)
    m1 = jnp.where(gt1, v, m1)
    vm1, vi1 = _bcast_max_minidx(m1, i1v, liota)
    in_l = i1v == vi1
    vl = jnp.where(in_l, m2, m1)
    il = jnp.where(in_l, i2v, i1v)
    vm2, vi2 = _bcast_max_minidx(vl, il, liota)
    return vm1, vi1, vm2, vi2


@functools.cache
def _topk_gather_kernel():
    mesh = plsc.VectorSubcoreMesh(core_axis_name="c", subcore_axis_name="s")

    @functools.partial(
        pl.kernel,
        mesh=mesh,
        out_type=(
            jax.ShapeDtypeStruct((BATCH, TOPK, PDIM), jnp.float32),
            jax.ShapeDtypeStruct((NW * NJOB, LANES), jnp.int32),
            jax.ShapeDtypeStruct((NW, LANES), jnp.float32),
        ),
        scratch_types=[
            pltpu.VMEM((BCH, POOL), jnp.float32),
            pltpu.VMEM((NJOB, LANES), jnp.int32),
            pltpu.VMEM((LANES,), jnp.float32),
            pltpu.VMEM((BCH, 1, PDIM), jnp.float32),
            pltpu.VMEM((BCH, 1, PDIM), jnp.float32),
            pltpu.SemaphoreType.DMA,
            pltpu.SemaphoreType.DMA,
            pltpu.SemaphoreType.DMA,
            pltpu.SemaphoreType.DMA,
        ],
    )
    def _topk_gather(sim_hbm, prompt_hbm, out_hbm, idx_hbm, sums_hbm,
                     simbuf, jobtab, saccbuf, buf_a, buf_b,
                     gsem_a, gsem_b, wsem_a, wsem_b):
        wid = lax.axis_index("s") * NC + lax.axis_index("c")
        bb = wid * BPW
        liota = lax.iota(jnp.int32, LANES)
        state = {"total": jnp.zeros((LANES,), jnp.float32)}

        def topk_chunk(c):
            pltpu.sync_copy(sim_hbm.at[pl.ds(bb + c * BCH, BCH)], simbuf)
            v1 = jnp.zeros((LANES,), jnp.int32)
            v2 = jnp.zeros((LANES,), jnp.int32)
            for rr in range(BCH):
                vm1, vi1, vm2, vi2 = _row_top2(simbuf, rr)
                v1 = jnp.where(liota == rr, vi1, v1)
                v2 = jnp.where(liota == rr, vi2, v2)
                state["total"] = state["total"] + jnp.where(
                    liota == 0, vm1 + vm2, 0.0
                )
            jobtab[2 * c + 0, :] = v1
            jobtab[2 * c + 1, :] = v2

        topk_chunk(0)
        topk_chunk(1)
        writes = []
        for j in range(NJOB):
            c, p = j // 2, j % 2
            buf, gsem, wsem = (
                (buf_a, gsem_a, wsem_a) if j % 2 == 0 else (buf_b, gsem_b, wsem_b)
            )
            if j >= 2:
                writes[j - 2].wait()
            gath = pltpu.async_copy(
                prompt_hbm.at[jobtab.at[j, pl.ds(0, BCH)]], buf, gsem
            )
            if p == 0 and c + 2 < NBC:
                topk_chunk(c + 2)
            gath.wait()
            writes.append(
                pltpu.async_copy(
                    buf,
                    out_hbm.at[pl.ds(bb + c * BCH, BCH), pl.ds(p, 1)],
                    wsem,
                )
            )
        saccbuf[...] = state["total"]
        pltpu.sync_copy(saccbuf, sums_hbm.at[wid])
        pltpu.sync_copy(jobtab, idx_hbm.at[pl.ds(wid * NJOB, NJOB)])
        writes[-2].wait()
        writes[-1].wait()

    return _topk_gather


def kernel(x_embed, prompt, prompt_key):
    pk_norm = _pk_norm(prompt_key)
    sim = _sim(x_embed, pk_norm)
    batched_prompt, jobs, sums = _topk_gather_kernel()(sim, prompt)
    idx = (
        jobs.reshape(NW, NBC, TOPK, LANES)[:, :, :, :BCH]
        .transpose(0, 1, 3, 2)
        .reshape(BATCH, TOPK)
    )
    reduce_sim = _rsum(sums)[0, 0]
    return batched_prompt, sim, idx, reduce_sim
